# baseline (speedup 1.0000x reference)
import dataclasses
import functools

import jax
import jax.numpy as jnp
from jax import lax
from jax.experimental import pallas as pl
from jax.experimental.pallas import tpu as pltpu
from jax.experimental.pallas import tpu_sc as plsc

NC = 2
NS = 16
NW = NC * NS
LANES = 16
CH = 128
NB = 2
NI = 4


def _sc_agg_make(N, D, E, with_deg):
    n_chunks = E // CH
    W = n_chunks // NW
    extra = n_chunks - W * NW
    U = 80
    n_units = N // U
    units_per_sub = (n_units + NS - 1) // NS

    out_type = [jax.ShapeDtypeStruct((NC, N, D), jnp.float32)]
    if with_deg:
        out_type.append(jax.ShapeDtypeStruct((NW, N), jnp.float32))

    scratch = [
        pltpu.VMEM((NI, 2, CH), jnp.int32),
        pltpu.VMEM((NB, CH, D), jnp.float32),
        pltpu.VMEM_SHARED((N, D), jnp.float32),
        pltpu.SemaphoreType.DMA,
        pltpu.SemaphoreType.DMA,
        pltpu.SemaphoreType.DMA,
        pltpu.SemaphoreType.DMA,
    ]
    if with_deg:
        scratch.append(pltpu.VMEM((N,), jnp.float32))

    mesh = plsc.VectorSubcoreMesh(core_axis_name="c", subcore_axis_name="s")

    def body(*refs):
        if with_deg:
            (x_hbm, edge_hbm, agg_hbm, deg_hbm,
             ib, rb, agg_sh, sem_i, sem_g0, sem_g1, sem_s, degp) = refs
        else:
            (x_hbm, edge_hbm, agg_hbm,
             ib, rb, agg_sh, sem_i, sem_g0, sem_g1, sem_s) = refs
        gsems = (sem_g0, sem_g1)

        c = lax.axis_index("c")
        s = lax.axis_index("s")
        wid = s * NC + c

        z16 = jnp.zeros((LANES,), jnp.float32)
        one16 = jnp.full((LANES,), 1.0, jnp.float32)

        stage = rb.at[0, pl.ds(0, U)]

        @pl.loop(0, U)
        def _(i):
            @pl.loop(0, D // LANES)
            def _(j):
                rb[0, i, pl.ds(j * LANES, LANES)] = z16

        if with_deg:
            @pl.loop(0, N // LANES)
            def _(i):
                degp[pl.ds(i * LANES, LANES)] = z16

        @pl.loop(0, units_per_sub)
        def _(t):
            unit = t * NS + s

            @pl.when(unit < n_units)
            def _():
                pltpu.sync_copy(stage, agg_sh.at[pl.ds(unit * U, U)])

        plsc.subcore_barrier()

        def ebase(k):
            return (k * NW + wid) * CH

        def idx_start(k, j):
            pltpu.make_async_copy(edge_hbm.at[:, pl.ds(ebase(k), CH)],
                                  ib.at[j], sem_i).start()

        def idx_wait(j):
            pltpu.make_async_copy(edge_hbm.at[:, pl.ds(0, CH)],
                                  ib.at[j], sem_i).wait()

        def gather_start(k, j, b):
            pltpu.make_async_copy(x_hbm.at[ib.at[j, 0]], rb.at[b],
                                  gsems[b]).start()

        def gather_wait(j, b):
            pltpu.make_async_copy(x_hbm.at[ib.at[j, 0]], rb.at[b],
                                  gsems[b]).wait()

        def scat_start(j, b):
            pltpu.async_copy(rb.at[b], agg_sh.at[ib.at[j, 1]], sem_s,
                             add=True)

        def scat_wait(j, b):
            pltpu.make_async_copy(rb.at[b], agg_sh.at[ib.at[j, 1]],
                                  sem_s).wait()

        def deg_update(j):
            if with_deg:
                @pl.loop(0, CH // LANES)
                def _(q):
                    idxr = ib[j, 1, pl.ds(q * LANES, LANES)]
                    plsc.addupdate_scatter(degp, [idxr], one16)

        def consume(j, b):
            pltpu.sync_copy(rb.at[b], agg_sh.at[ib.at[j, 1]], add=True)
            deg_update(j)

        def item(k):
            b, j = k % NB, k % NI
            if k + 1 < W:
                idx_wait((k + 1) % NI)
                if k >= 1:
                    scat_wait((k - 1) % NI, (k - 1) % NB)
                gather_start(k + 1, (k + 1) % NI, (k + 1) % NB)
                if k + 2 < W:
                    idx_start(k + 2, (k + 2) % NI)
                gather_wait(j, b)
                scat_start(j, b)
                deg_update(j)
            else:
                scat_wait((k - 1) % NI, (k - 1) % NB)
                gather_wait(j, b)
                scat_start(j, b)
                deg_update(j)
                scat_wait(j, b)

        idx_start(0, 0)
        idx_wait(0)
        gather_start(0, 0, 0)
        idx_start(1, 1)

        item(0)
        n_unrolled = ((W - 2 - 1) // NI) * NI

        @pl.loop(0, n_unrolled // NI)
        def _(t):
            k0 = t * NI + 1
            for d in range(NI):
                k_ph = (1 + d)
                b, j = k_ph % NB, k_ph % NI
                idx_wait((k_ph + 1) % NI)
                scat_wait((k_ph - 1) % NI, (k_ph - 1) % NB)
                pltpu.make_async_copy(
                    x_hbm.at[ib.at[(k_ph + 1) % NI, 0]],
                    rb.at[(k_ph + 1) % NB], gsems[(k_ph + 1) % NB]).start()
                pltpu.make_async_copy(
                    edge_hbm.at[:, pl.ds(ebase(k0 + d + 2), CH)],
                    ib.at[(k_ph + 2) % NI], sem_i).start()
                gather_wait(j, b)
                scat_start(j, b)
                deg_update(j)

        for k in range(n_unrolled + 1, W):
            item(k)

        if extra:
            @pl.when(wid < extra)
            def _():
                base = (W * NW + wid) * CH
                pltpu.make_async_copy(edge_hbm.at[:, pl.ds(base, CH)],
                                      ib.at[0], sem_i).start()
                idx_wait(0)
                gather_start(0, 0, 0)
                gather_wait(0, 0)
                consume(0, 0)

        plsc.subcore_barrier()

        @pl.loop(0, units_per_sub)
        def _(t):
            unit = t * NS + s

            @pl.when(unit < n_units)
            def _():
                pltpu.sync_copy(agg_sh.at[pl.ds(unit * U, U)], stage)
                pltpu.sync_copy(stage, agg_hbm.at[c, pl.ds(unit * U, U)])

        if with_deg:
            pltpu.sync_copy(degp, deg_hbm.at[wid])

    cp = pltpu.CompilerParams()
    if "needs_layout_passes" in pltpu.CompilerParams.__dataclass_fields__:
        cp = dataclasses.replace(cp, needs_layout_passes=False)
    return pl.kernel(body, out_type=out_type, mesh=mesh,
                     scratch_types=scratch, compiler_params=cp)


def _combine_body(agg_ref, deg_ref, x_ref, wl_ref, wr_ref, b_ref, o_ref,
                  *, relu):
    a = agg_ref[0] + agg_ref[1]
    deg = jnp.sum(deg_ref[...], axis=1, keepdims=True)
    mean = a * (1.0 / jnp.maximum(deg, 1.0))
    acc = lax.dot_general(mean, wl_ref[...], (((1,), (1,)), ((), ())),
                          preferred_element_type=jnp.float32)
    acc = acc + lax.dot_general(x_ref[...], wr_ref[...],
                                (((1,), (1,)), ((), ())),
                                      preferred_element_type=jnp.float32)
    acc = acc + b_ref[...]
    if relu:
        acc = jnp.maximum(acc, 0.0)
    o_ref[...] = acc


def _combine_make(N, D, relu, R=None):
    R = R or N
    return pl.pallas_call(
        functools.partial(_combine_body, relu=relu),
        grid=(N // R,),
        in_specs=[
            pl.BlockSpec((NC, R, D), lambda i: (0, i, 0)),
            pl.BlockSpec((R, NW), lambda i: (i, 0)),
            pl.BlockSpec((R, D), lambda i: (i, 0)),
            pl.BlockSpec((D, D), lambda i: (0, 0)),
            pl.BlockSpec((D, D), lambda i: (0, 0)),
            pl.BlockSpec((1, D), lambda i: (0, 0)),
        ],
        out_specs=pl.BlockSpec((R, D), lambda i: (i, 0)),
        out_shape=jax.ShapeDtypeStruct((N, D), jnp.float32),
    )


@functools.lru_cache(maxsize=None)
def _build(N, D, E):
    sc_agg_deg = _sc_agg_make(N, D, E, with_deg=True)
    sc_agg = _sc_agg_make(N, D, E, with_deg=False)
    combine_relu = _combine_make(N, D, relu=True)
    combine = _combine_make(N, D, relu=False)
    return sc_agg_deg, sc_agg, combine_relu, combine


def kernel(x, edge_index, W1_l, b1, W1_r, W2_l, b2, W2_r):
    N, D = x.shape
    E = edge_index.shape[1]
    sc_agg_deg, sc_agg, combine_relu, combine = _build(N, D, E)

    agg_x, deg_p = sc_agg_deg(x, edge_index)
    deg_t = deg_p.T
    h = combine_relu(agg_x, deg_t, x, W1_l, W1_r, b1.reshape(1, D))
    (agg_h,) = sc_agg(h, edge_index)
    out = combine(agg_h, deg_t, h, W2_l, W2_r, b2.reshape(1, D))
    return out


# speedup vs baseline: 1.0029x; 1.0029x over previous
import dataclasses
import functools

import jax
import jax.numpy as jnp
from jax import lax
from jax.experimental import pallas as pl
from jax.experimental.pallas import tpu as pltpu
from jax.experimental.pallas import tpu_sc as plsc

NC = 2
NS = 16
NW = NC * NS
LANES = 16
CH = 128
NB = 2
NI = 4


def _sc_agg_make(N, D, E, with_deg):
    n_chunks = E // CH
    W = n_chunks // NW
    extra = n_chunks - W * NW
    U = 80
    n_units = N // U
    units_per_sub = (n_units + NS - 1) // NS

    out_type = [jax.ShapeDtypeStruct((NC, N, D), jnp.float32)]
    if with_deg:
        out_type.append(jax.ShapeDtypeStruct((NW, N), jnp.float32))

    scratch = [
        pltpu.VMEM((NI, 2, CH), jnp.int32),
        pltpu.VMEM((NB, CH, D), jnp.float32),
        pltpu.VMEM_SHARED((N, D), jnp.float32),
        pltpu.SemaphoreType.DMA,
        pltpu.SemaphoreType.DMA,
        pltpu.SemaphoreType.DMA,
        pltpu.SemaphoreType.DMA,
    ]
    if with_deg:
        scratch.append(pltpu.VMEM((N,), jnp.float32))

    mesh = plsc.VectorSubcoreMesh(core_axis_name="c", subcore_axis_name="s")

    def body(*refs):
        if with_deg:
            (x_hbm, edge_hbm, agg_hbm, deg_hbm,
             ib, rb, agg_sh, sem_i, sem_g0, sem_g1, sem_s, degp) = refs
        else:
            (x_hbm, edge_hbm, agg_hbm,
             ib, rb, agg_sh, sem_i, sem_g0, sem_g1, sem_s) = refs
        gsems = (sem_g0, sem_g1)

        c = lax.axis_index("c")
        s = lax.axis_index("s")
        wid = s * NC + c

        z16 = jnp.zeros((LANES,), jnp.float32)
        one16 = jnp.full((LANES,), 1.0, jnp.float32)

        stage = rb.at[0, pl.ds(0, U)]

        @pl.loop(0, U)
        def _(i):
            @pl.loop(0, D // LANES)
            def _(j):
                rb[0, i, pl.ds(j * LANES, LANES)] = z16

        if with_deg:
            @pl.loop(0, N // LANES)
            def _(i):
                degp[pl.ds(i * LANES, LANES)] = z16

        @pl.loop(0, units_per_sub)
        def _(t):
            unit = t * NS + s

            @pl.when(unit < n_units)
            def _():
                pltpu.sync_copy(stage, agg_sh.at[pl.ds(unit * U, U)])

        plsc.subcore_barrier()

        def ebase(k):
            return (k * NW + wid) * CH

        def idx_start(k, j):
            pltpu.make_async_copy(edge_hbm.at[:, pl.ds(ebase(k), CH)],
                                  ib.at[j], sem_i).start()

        def idx_wait(j):
            pltpu.make_async_copy(edge_hbm.at[:, pl.ds(0, CH)],
                                  ib.at[j], sem_i).wait()

        def gather_start(k, j, b):
            pltpu.make_async_copy(x_hbm.at[ib.at[j, 0]], rb.at[b],
                                  gsems[b]).start()

        def gather_wait(j, b):
            pltpu.make_async_copy(x_hbm.at[ib.at[j, 0]], rb.at[b],
                                  gsems[b]).wait()

        def scat_start(j, b):
            pltpu.async_copy(rb.at[b], agg_sh.at[ib.at[j, 1]], sem_s,
                             add=True)

        def scat_wait(j, b):
            pltpu.make_async_copy(rb.at[b], agg_sh.at[ib.at[j, 1]],
                                  sem_s).wait()

        def deg_update(j):
            if with_deg:
                @pl.loop(0, CH // LANES)
                def _(q):
                    idxr = ib[j, 1, pl.ds(q * LANES, LANES)]
                    plsc.addupdate_scatter(degp, [idxr], one16)

        def consume(j, b):
            pltpu.sync_copy(rb.at[b], agg_sh.at[ib.at[j, 1]], add=True)
            deg_update(j)

        def item(k):
            b, j = k % NB, k % NI
            if k + 1 < W:
                idx_wait((k + 1) % NI)
                if k >= 1:
                    scat_wait((k - 1) % NI, (k - 1) % NB)
                gather_start(k + 1, (k + 1) % NI, (k + 1) % NB)
                if k + 2 < W:
                    idx_start(k + 2, (k + 2) % NI)
                gather_wait(j, b)
                scat_start(j, b)
                deg_update(j)
            else:
                scat_wait((k - 1) % NI, (k - 1) % NB)
                gather_wait(j, b)
                scat_start(j, b)
                deg_update(j)
                scat_wait(j, b)

        idx_start(0, 0)
        idx_wait(0)
        gather_start(0, 0, 0)
        idx_start(1, 1)

        item(0)
        n_unrolled = ((W - 2 - 1) // NI) * NI

        @pl.loop(0, n_unrolled // NI)
        def _(t):
            k0 = t * NI + 1
            for d in range(NI):
                k_ph = (1 + d)
                b, j = k_ph % NB, k_ph % NI
                idx_wait((k_ph + 1) % NI)
                scat_wait((k_ph - 1) % NI, (k_ph - 1) % NB)
                pltpu.make_async_copy(
                    x_hbm.at[ib.at[(k_ph + 1) % NI, 0]],
                    rb.at[(k_ph + 1) % NB], gsems[(k_ph + 1) % NB]).start()
                pltpu.make_async_copy(
                    edge_hbm.at[:, pl.ds(ebase(k0 + d + 2), CH)],
                    ib.at[(k_ph + 2) % NI], sem_i).start()
                gather_wait(j, b)
                scat_start(j, b)
                deg_update(j)

        for k in range(n_unrolled + 1, W):
            item(k)

        if extra:
            @pl.when(wid < extra)
            def _():
                base = (W * NW + wid) * CH
                pltpu.make_async_copy(edge_hbm.at[:, pl.ds(base, CH)],
                                      ib.at[0], sem_i).start()
                idx_wait(0)
                gather_start(0, 0, 0)
                gather_wait(0, 0)
                consume(0, 0)

        plsc.subcore_barrier()

        @pl.loop(0, units_per_sub)
        def _(t):
            unit = t * NS + s

            @pl.when(unit < n_units)
            def _():
                pltpu.sync_copy(agg_sh.at[pl.ds(unit * U, U)], stage)
                pltpu.sync_copy(stage, agg_hbm.at[c, pl.ds(unit * U, U)])

        if with_deg:
            pltpu.sync_copy(degp, deg_hbm.at[wid])

    cp = pltpu.CompilerParams()
    if "needs_layout_passes" in pltpu.CompilerParams.__dataclass_fields__:
        cp = dataclasses.replace(cp, needs_layout_passes=False)
    return pl.kernel(body, out_type=out_type, mesh=mesh,
                     scratch_types=scratch, compiler_params=cp)


def _combine_body(agg_ref, deg_ref, x_ref, wl_ref, wr_ref, b_ref, o_ref,
                  *, relu):
    a = agg_ref[0] + agg_ref[1]
    deg = jnp.sum(deg_ref[...], axis=1, keepdims=True)
    mean = a * (1.0 / jnp.maximum(deg, 1.0))
    acc = lax.dot_general(mean, wl_ref[...], (((1,), (1,)), ((), ())),
                          preferred_element_type=jnp.float32)
    acc = acc + lax.dot_general(x_ref[...], wr_ref[...],
                                (((1,), (1,)), ((), ())),
                                      preferred_element_type=jnp.float32)
    acc = acc + b_ref[...]
    if relu:
        acc = jnp.maximum(acc, 0.0)
    o_ref[...] = acc


def _combine_make(N, D, relu, R=2000):
    return pl.pallas_call(
        functools.partial(_combine_body, relu=relu),
        grid=(N // R,),
        in_specs=[
            pl.BlockSpec((NC, R, D), lambda i: (0, i, 0)),
            pl.BlockSpec((R, NW), lambda i: (i, 0)),
            pl.BlockSpec((R, D), lambda i: (i, 0)),
            pl.BlockSpec((D, D), lambda i: (0, 0)),
            pl.BlockSpec((D, D), lambda i: (0, 0)),
            pl.BlockSpec((1, D), lambda i: (0, 0)),
        ],
        out_specs=pl.BlockSpec((R, D), lambda i: (i, 0)),
        out_shape=jax.ShapeDtypeStruct((N, D), jnp.float32),
    )


@functools.lru_cache(maxsize=None)
def _build(N, D, E):
    sc_agg_deg = _sc_agg_make(N, D, E, with_deg=True)
    sc_agg = _sc_agg_make(N, D, E, with_deg=False)
    combine_relu = _combine_make(N, D, relu=True)
    combine = _combine_make(N, D, relu=False)
    return sc_agg_deg, sc_agg, combine_relu, combine


def kernel(x, edge_index, W1_l, b1, W1_r, W2_l, b2, W2_r):
    N, D = x.shape
    E = edge_index.shape[1]
    sc_agg_deg, sc_agg, combine_relu, combine = _build(N, D, E)

    agg_x, deg_p = sc_agg_deg(x, edge_index)
    deg_t = deg_p.T
    h = combine_relu(agg_x, deg_t, x, W1_l, W1_r, b1.reshape(1, D))
    (agg_h,) = sc_agg(h, edge_index)
    out = combine(agg_h, deg_t, h, W2_l, W2_r, b2.reshape(1, D))
    return out
